# baseline (speedup 1.0000x reference)
import jax
import jax.numpy as jnp
from jax import lax
from jax.experimental import pallas as pl
from jax.experimental.pallas import tpu as pltpu
from jax.experimental.pallas import tpu_sc as plsc

_NL = 8
_SIG = 0.5
_NW = 32
_CHUNK = 512 * 512 // _NW


def _sc_sums(km_h, kl_h, rl_h, p_h, out_h, klv, rlv, kmv, pv, hist):
    wid = lax.axis_index("s") * 2 + lax.axis_index("c")
    base = wid * _CHUNK
    for img in range(4):
        for j in range(48):
            hist[img, j] = jnp.zeros((16,), jnp.float32)
    lanes = lax.iota(jnp.int32, 16)
    for img in range(4):
        pltpu.sync_copy(kl_h.at[img, pl.ds(base, _CHUNK)], klv)
        pltpu.sync_copy(rl_h.at[img, pl.ds(base, _CHUNK)], rlv)
        pltpu.sync_copy(km_h.at[img, pl.ds(base, _CHUNK)], kmv)
        for c in range(4):
            pltpu.sync_copy(p_h.at[img, c, pl.ds(base, _CHUNK)], pv.at[c])
        imgs = jnp.full((16,), img, jnp.int32)

        def body(i, _):
            off = i * 16
            kle = klv[pl.ds(off, 16)]
            rle = rlv[pl.ds(off, 16)]
            kme = kmv[pl.ds(off, 16)]
            plsc.addupdate_scatter(hist, [imgs, kle, lanes], kme)
            for c in range(4):
                pe = pv[c, pl.ds(off, 16)]
                plsc.addupdate_scatter(hist, [imgs, kle + 8 * (1 + c), lanes], pe)
            plsc.addupdate_scatter(hist, [imgs, rle + 40, lanes], kme)
            return 0

        lax.fori_loop(0, _CHUNK // 16, body, 0)
    pltpu.sync_copy(hist, out_h.at[wid])


def _tc_body(hist_ref, pred_ref, rm_ref, rl_ref, kl_ref, loss_ref):
    b = pl.program_id(0)
    nb = pl.num_programs(0)
    kl = kl_ref[0, 0]
    rl = rl_ref[0, 0]
    rm = rm_ref[0, 0]
    H = kl.shape[0]

    t48 = jnp.sum(hist_ref[...], axis=(0, 1, 3))[None, :]
    lane = jax.lax.broadcasted_iota(jnp.int32, (1, _NL), 1)
    inv_k = 1.0 / (t48[:, 0:_NL] + 1.0)
    g_t = [jnp.where(lane > 0, t48[:, _NL * (1 + c):_NL * (2 + c)] * inv_k, 0.0)
           for c in range(4)]
    rinv_t = jnp.where(lane > 0, 1.0 / (t48[:, 40:48] + 1.0), 1.0)

    def gather(t, idx):
        tb = jnp.broadcast_to(t, (H, _NL))
        return jnp.take_along_axis(tb, idx, axis=1, mode="promise_in_bounds")

    acc = jnp.zeros_like(rm)
    for c in range(4):
        fp = pred_ref[0, c] * rm
        d = fp - gather(g_t[c], kl)
        acc = acc + d * d
    dd = jnp.maximum(jnp.sqrt(acc) - _SIG, 0.0)
    dd = jnp.log(dd * dd + 1.0)
    s = jnp.sum(dd * gather(rinv_t, rl))

    @pl.when(b == 0)
    def _():
        loss_ref[0, 0] = s

    @pl.when(jnp.logical_and(b != 0, b != nb - 1))
    def _():
        loss_ref[0, 0] = loss_ref[0, 0] + s

    @pl.when(jnp.logical_and(b != 0, b == nb - 1))
    def _():
        numk = jnp.max(kl).astype(jnp.float32)
        loss_ref[0, 0] = (loss_ref[0, 0] + s) / numk


def kernel(pred_similarities, regions_mask, kernels_mask, text_mask_ndi_labels, kernel_mask_ndi_labels):
    B, C, H, W = pred_similarities.shape
    N = H * W

    km2 = kernels_mask.reshape(B, N)
    kl2 = kernel_mask_ndi_labels.reshape(B, N)
    rl2 = text_mask_ndi_labels.reshape(B, N)
    p2 = pred_similarities.reshape(B, C, N)

    mesh = plsc.VectorSubcoreMesh(core_axis_name="c", subcore_axis_name="s")
    hist = pl.kernel(
        _sc_sums,
        out_type=jax.ShapeDtypeStruct((_NW, B, 48, 16), jnp.float32),
        mesh=mesh,
        compiler_params=pltpu.CompilerParams(needs_layout_passes=False),
        scratch_types=[
            pltpu.VMEM((_CHUNK,), jnp.int32),
            pltpu.VMEM((_CHUNK,), jnp.int32),
            pltpu.VMEM((_CHUNK,), jnp.float32),
            pltpu.VMEM((C, _CHUNK), jnp.float32),
            pltpu.VMEM((B, 48, 16), jnp.float32),
        ],
    )(km2, kl2, rl2, p2)

    img_spec = lambda: pl.BlockSpec((1, 1, H, W), lambda b: (b, 0, 0, 0))
    loss = pl.pallas_call(
        _tc_body,
        grid=(B,),
        in_specs=[
            pl.BlockSpec((_NW, 1, 48, 16), lambda b: (0, b, 0, 0)),
            pl.BlockSpec((1, C, H, W), lambda b: (b, 0, 0, 0)),
            img_spec(),
            img_spec(),
            img_spec(),
        ],
        out_specs=pl.BlockSpec(memory_space=pltpu.SMEM),
        out_shape=jax.ShapeDtypeStruct((1, 1), jnp.float32),
    )(hist, pred_similarities, regions_mask, text_mask_ndi_labels, kernel_mask_ndi_labels)

    return loss[0, 0]


# speedup vs baseline: 4.7963x; 4.7963x over previous
import jax
import jax.numpy as jnp
from jax.experimental import pallas as pl
from jax.experimental.pallas import tpu as pltpu

_NL = 8
_SIG = 0.5


def _body(pred_ref, rm_ref, km_ref, rl_ref, kl_ref, loss_ref):
    b = pl.program_id(0)
    nb = pl.num_programs(0)
    kl = kl_ref[0, 0]
    rl = rl_ref[0, 0]
    km = km_ref[0, 0]
    rm = rm_ref[0, 0]
    H = kl.shape[0]
    preds = [pred_ref[0, c] for c in range(4)]

    z11 = jnp.zeros((1, 1), jnp.float32)

    def msum(mask, data):
        return jnp.sum(jnp.where(mask, data, 0.0), axis=(0, 1), keepdims=True)

    kmask = [kl == l for l in range(1, _NL)]
    rmask = [rl == l for l in range(1, _NL)]
    ks_t = jnp.concatenate([z11] + [msum(m, km) for m in kmask], axis=0)
    rs_t = jnp.concatenate([z11] + [msum(m, km) for m in rmask], axis=0)
    cs_t = [jnp.concatenate([z11] + [msum(m, p) for m in kmask], axis=0)
            for p in preds]

    inv_k = 1.0 / (ks_t + 1.0)
    g_t = [c * inv_k for c in cs_t]
    lane = jax.lax.broadcasted_iota(jnp.int32, (_NL, 1), 0)
    rinv_t = jnp.where(lane > 0, 1.0 / (rs_t + 1.0), 1.0)

    W = kl.shape[1]

    def gather(t, idx):
        tb = jnp.broadcast_to(t, (_NL, W))
        return jnp.take_along_axis(tb, idx, axis=0, mode="promise_in_bounds")

    acc = jnp.zeros_like(km)
    for c in range(4):
        fp = preds[c] * rm
        d = fp - gather(g_t[c], kl)
        acc = acc + d * d
    dd = jnp.maximum(jnp.sqrt(acc) - _SIG, 0.0)
    dd = jnp.log(dd * dd + 1.0)
    s = jnp.sum(dd * gather(rinv_t, rl))

    @pl.when(b == 0)
    def _():
        loss_ref[0, 0] = s

    @pl.when(jnp.logical_and(b != 0, b != nb - 1))
    def _():
        loss_ref[0, 0] = loss_ref[0, 0] + s

    @pl.when(jnp.logical_and(b != 0, b == nb - 1))
    def _():
        numk = jnp.max(kl).astype(jnp.float32)
        loss_ref[0, 0] = (loss_ref[0, 0] + s) / numk


def kernel(pred_similarities, regions_mask, kernels_mask, text_mask_ndi_labels, kernel_mask_ndi_labels):
    B, C, H, W = pred_similarities.shape

    img_spec = lambda: pl.BlockSpec((1, 1, H, W), lambda b: (b, 0, 0, 0))

    loss = pl.pallas_call(
        _body,
        grid=(B,),
        in_specs=[
            pl.BlockSpec((1, C, H, W), lambda b: (b, 0, 0, 0)),
            img_spec(),
            img_spec(),
            img_spec(),
            img_spec(),
        ],
        out_specs=pl.BlockSpec(memory_space=pltpu.SMEM),
        out_shape=jax.ShapeDtypeStruct((1, 1), jnp.float32),
    )(pred_similarities, regions_mask, kernels_mask, text_mask_ndi_labels, kernel_mask_ndi_labels)

    return loss[0, 0]
